# Initial kernel scaffold; baseline (speedup 1.0000x reference)
import jax
import jax.numpy as jnp
from jax.experimental import pallas as pl


def kernel(inputs, node_feature, edge_index, W1, b1, W2, b2):
    raise NotImplementedError("write your pallas kernel here")


# speedup vs baseline: 3.0459x; 3.0459x over previous
import functools

import jax
import jax.numpy as jnp
from jax import lax
from jax.experimental import pallas as pl
from jax.experimental.pallas import tpu as pltpu
from jax.experimental.pallas import tpu_sc as plsc

N_NODES = 10000
N_EDGES = 320000
D = 128
BATCH = 4096

NC = 2
NS = 16
NW = NC * NS
CHUNK = 128
G = 8
NG = 10
NCH = G * NG
E_PAD = NW * NCH * CHUNK
NP = 10240
ROWS_PER_TILE = NP // NS
TRASH = N_NODES


def _make_agg(with_deg: bool):
    mesh = plsc.VectorSubcoreMesh(core_axis_name="c", subcore_axis_name="s")
    out_type = [jax.ShapeDtypeStruct((NC, NP, D), jnp.float32)]
    scratch = [
        pltpu.VMEM((CHUNK,), jnp.int32),
        pltpu.VMEM((CHUNK,), jnp.int32),
        pltpu.VMEM((CHUNK, D), jnp.float32),
        pltpu.VMEM_SHARED((NP, D), jnp.float32),
        pltpu.SemaphoreType.DMA,
    ]
    if with_deg:
        out_type.append(jax.ShapeDtypeStruct((NC, NP), jnp.float32))
        scratch += [
            pltpu.VMEM((CHUNK,), jnp.float32),
            pltpu.VMEM_SHARED((NP,), jnp.float32),
        ]

    def body(x_hbm, src_hbm, dst_hbm, zeros_hbm, dzeros_hbm, ones_hbm,
             out_p, *rest):
        if with_deg:
            out_deg, src_v, dst_v, gbuf, agg_sh, sem, ones_v, deg_sh = rest
        else:
            src_v, dst_v, gbuf, agg_sh, sem = rest
        c = lax.axis_index("c")
        s = lax.axis_index("s")
        wid = c * NS + s
        pltpu.sync_copy(zeros_hbm, agg_sh.at[pl.ds(s * ROWS_PER_TILE, ROWS_PER_TILE)])
        if with_deg:
            pltpu.sync_copy(dzeros_hbm,
                            deg_sh.at[pl.ds(s * ROWS_PER_TILE, ROWS_PER_TILE)])
            pltpu.sync_copy(ones_hbm, ones_v)
        plsc.subcore_barrier()

        def chunk_body(j, carry):
            pltpu.sync_copy(src_hbm.at[wid, j], src_v)
            pltpu.sync_copy(dst_hbm.at[wid, j], dst_v)
            pltpu.async_copy(x_hbm.at[src_v], gbuf, sem).wait()
            pltpu.sync_copy(gbuf, agg_sh.at[dst_v], add=True)
            if with_deg:
                pltpu.sync_copy(ones_v, deg_sh.at[dst_v], add=True)
            return carry

        lax.fori_loop(0, NCH, chunk_body, 0)
        plsc.subcore_barrier()
        sl = pl.ds(s * ROWS_PER_TILE, ROWS_PER_TILE)
        pltpu.sync_copy(agg_sh.at[sl], out_p.at[c, sl])
        if with_deg:
            pltpu.sync_copy(deg_sh.at[sl], out_deg.at[c, sl])

    return pl.kernel(body, out_type=tuple(out_type) if with_deg else out_type[0],
                     mesh=mesh, scratch_types=tuple(scratch))


_agg_deg = _make_agg(True)
_agg = _make_agg(False)


def _tc_layer1_body(p_ref, degp_ref, x_ref, w_ref, b_ref, h_ref, degc_ref):
    agg = p_ref[0] + p_ref[1]
    deg = degp_ref[0] + degp_ref[1]
    degc = jnp.maximum(deg, 1.0)
    h = agg / degc[:, None]
    y = jnp.dot(h, w_ref[...], preferred_element_type=jnp.float32) + b_ref[...]
    h_ref[...] = jnp.maximum(y, 0.0) + x_ref[...]
    degc_ref[...] = degc


def _tc_layer2_body(p_ref, degc_ref, x_ref, w_ref, b_ref, h_ref):
    agg = p_ref[0] + p_ref[1]
    h = agg / degc_ref[...][:, None]
    y = jnp.dot(h, w_ref[...], preferred_element_type=jnp.float32) + b_ref[...]
    h_ref[...] = jnp.maximum(y, 0.0) + x_ref[...]


_R = 2048


def _tc_layer1(p, degp, x, w, b):
    grid = (NP // _R,)
    return pl.pallas_call(
        _tc_layer1_body,
        grid=grid,
        in_specs=[
            pl.BlockSpec((NC, _R, D), lambda i: (0, i, 0)),
            pl.BlockSpec((NC, _R), lambda i: (0, i)),
            pl.BlockSpec((_R, D), lambda i: (i, 0)),
            pl.BlockSpec((D, D), lambda i: (0, 0)),
            pl.BlockSpec((D,), lambda i: (0,)),
        ],
        out_specs=[
            pl.BlockSpec((_R, D), lambda i: (i, 0)),
            pl.BlockSpec((_R,), lambda i: (i,)),
        ],
        out_shape=[
            jax.ShapeDtypeStruct((NP, D), jnp.float32),
            jax.ShapeDtypeStruct((NP,), jnp.float32),
        ],
    )(p, degp, x, w, b)


def _tc_layer2(p, degc, x, w, b):
    grid = (NP // _R,)
    return pl.pallas_call(
        _tc_layer2_body,
        grid=grid,
        in_specs=[
            pl.BlockSpec((NC, _R, D), lambda i: (0, i, 0)),
            pl.BlockSpec((_R,), lambda i: (i,)),
            pl.BlockSpec((_R, D), lambda i: (i, 0)),
            pl.BlockSpec((D, D), lambda i: (0, 0)),
            pl.BlockSpec((D,), lambda i: (0,)),
        ],
        out_specs=pl.BlockSpec((_R, D), lambda i: (i, 0)),
        out_shape=jax.ShapeDtypeStruct((NP, D), jnp.float32),
    )(p, degc, x, w, b)


PAIRS_PER_TILE = BATCH // NW


def _head_body(h_hbm, i1_hbm, i2_hbm, out_hbm, i1_v, i2_v, b1_v, b2_v,
               ob2_v, sem):
    c = lax.axis_index("c")
    s = lax.axis_index("s")
    wid = c * NS + s
    pltpu.sync_copy(i1_hbm.at[wid], i1_v)
    pltpu.sync_copy(i2_hbm.at[wid], i2_v)
    pltpu.async_copy(h_hbm.at[i1_v], b1_v, sem).wait()
    pltpu.async_copy(h_hbm.at[i2_v], b2_v, sem).wait()

    def pbody(p, carry):
        for k in range(D // 16):
            v1 = b1_v[p, pl.ds(k * 16, 16)]
            v2 = b2_v[p, pl.ds(k * 16, 16)]
            ob2_v[p, pl.ds(k * 16, 16)] = v1 * v2
        return carry

    lax.fori_loop(0, PAIRS_PER_TILE, pbody, 0)
    pltpu.sync_copy(ob2_v, out_hbm.at[pl.ds(wid * PAIRS_PER_TILE, PAIRS_PER_TILE)])


_head = pl.kernel(
    _head_body,
    out_type=jax.ShapeDtypeStruct((BATCH, D), jnp.float32),
    mesh=plsc.VectorSubcoreMesh(core_axis_name="c", subcore_axis_name="s"),
    scratch_types=(
        pltpu.VMEM((PAIRS_PER_TILE,), jnp.int32),
        pltpu.VMEM((PAIRS_PER_TILE,), jnp.int32),
        pltpu.VMEM((PAIRS_PER_TILE, D), jnp.float32),
        pltpu.VMEM((PAIRS_PER_TILE, D), jnp.float32),
        pltpu.VMEM((PAIRS_PER_TILE, D), jnp.float32),
        pltpu.SemaphoreType.DMA,
    ),
)


def _tc_reduce_body(ps_ref, out_ref):
    out_ref[...] = jnp.sum(ps_ref[...], axis=-1)


def _tc_reduce(ps):
    return pl.pallas_call(
        _tc_reduce_body,
        out_shape=jax.ShapeDtypeStruct((BATCH,), jnp.float32),
    )(ps)


def kernel(inputs, node_feature, edge_index, W1, b1, W2, b2):
    ei = edge_index.astype(jnp.int32)
    pad = E_PAD - N_EDGES
    src3 = jnp.concatenate([ei[0], jnp.zeros((pad,), jnp.int32)]).reshape(NW, NCH, CHUNK)
    dst3 = jnp.concatenate([ei[1], jnp.full((pad,), TRASH, jnp.int32)]).reshape(NW, NCH, CHUNK)
    x0 = jnp.pad(node_feature, ((0, NP - N_NODES), (0, 0)))
    zeros_h = jnp.zeros((ROWS_PER_TILE, D), jnp.float32)
    dzeros_h = jnp.zeros((ROWS_PER_TILE,), jnp.float32)
    ones_h = jnp.ones((CHUNK,), jnp.float32)

    p1, degp = _agg_deg(x0, src3, dst3, zeros_h, dzeros_h, ones_h)
    h1, degc = _tc_layer1(p1, degp, x0, W1, b1)
    p2 = _agg(h1, src3, dst3, zeros_h, dzeros_h, ones_h)
    h2 = _tc_layer2(p2, degc, h1, W2, b2)

    i1 = inputs[:, 0].astype(jnp.int32).reshape(NW, PAIRS_PER_TILE)
    i2 = inputs[:, 1].astype(jnp.int32).reshape(NW, PAIRS_PER_TILE)
    ps = _head(h2, i1, i2)
    return _tc_reduce(ps)
